# baseline (speedup 1.0000x reference)
import functools
import math

import jax
import jax.numpy as jnp
from jax import lax
from jax.experimental import pallas as pl
from jax.experimental.pallas import tpu as pltpu
from jax.experimental.pallas import tpu_sc as plsc

D_MODEL = 128
SCALE = math.sqrt(float(D_MODEL))

_info = plsc.get_sparse_core_info()
_NC, _NS, _L = _info.num_cores, _info.num_subcores, _info.num_lanes
_NW = _NC * _NS

CHUNK = 128
NBUF = 4


@functools.lru_cache(maxsize=None)
def _make_kernel(n_idx: int):
    assert n_idx % (_NW * CHUNK * NBUF) == 0
    b_per_w = n_idx // _NW
    n_chunks = b_per_w // CHUNK
    n_groups = n_chunks // NBUF
    mesh = plsc.VectorSubcoreMesh(core_axis_name="c", subcore_axis_name="s")

    scratch = [pltpu.VMEM((n_chunks, CHUNK), jnp.int32)]
    scratch += [pltpu.VMEM((CHUNK, D_MODEL), jnp.float32) for _ in range(NBUF)]
    scratch += [pltpu.SemaphoreType.DMA for _ in range(2 * NBUF)]

    @functools.partial(
        pl.kernel,
        mesh=mesh,
        out_type=jax.ShapeDtypeStruct((n_idx, D_MODEL), jnp.float32),
        scratch_types=scratch,
    )
    def emb(x_hbm, table_hbm, out_hbm, idx_v, *bufs_and_sems):
        bufs = bufs_and_sems[:NBUF]
        gsems = bufs_and_sems[NBUF:2 * NBUF]
        ssems = bufs_and_sems[2 * NBUF:]
        wid = lax.axis_index("s") * _NC + lax.axis_index("c")
        base = wid * b_per_w
        pltpu.sync_copy(x_hbm.at[pl.ds(wid * n_chunks, n_chunks)], idx_v)

        for b in range(NBUF):
            pltpu.async_copy(table_hbm.at[idx_v.at[b]], bufs[b], gsems[b])

        def group(g, carry):
            for b in range(NBUF):
                c = g * NBUF + b
                rows_v = bufs[b]
                pltpu.make_async_copy(
                    table_hbm.at[idx_v.at[b]], rows_v, gsems[b]).wait()

                def row_body(r, carry2, rows_v=rows_v):
                    for rr in range(2):
                        for j in range(D_MODEL // _L):
                            sl = pl.ds(j * _L, _L)
                            rows_v[2 * r + rr, sl] = (
                                rows_v[2 * r + rr, sl] * SCALE)
                    return carry2

                lax.fori_loop(0, CHUNK // 2, row_body, 0)
                store = pltpu.async_copy(
                    rows_v, out_hbm.at[pl.ds(base + c * CHUNK, CHUNK)],
                    ssems[b])

                @pl.when(g < n_groups - 1)
                def _refire(store=store, b=b, rows_v=rows_v, c=c):
                    store.wait()
                    pltpu.async_copy(
                        table_hbm.at[idx_v.at[c + NBUF]], rows_v, gsems[b])

            return carry

        lax.fori_loop(0, n_groups, group, 0)

        for b in range(NBUF):
            pltpu.make_async_copy(
                bufs[b], out_hbm.at[pl.ds(0, CHUNK)], ssems[b]).wait()

    return emb


def kernel(x, table):
    orig_shape = x.shape
    n_idx = x.size
    xf = x.reshape(n_idx // CHUNK, CHUNK).astype(jnp.int32)
    out = _make_kernel(n_idx)(xf, table)
    return out.reshape(*orig_shape, D_MODEL)
